# baseline (speedup 1.0000x reference)
import functools

import jax
import jax.numpy as jnp
from jax import lax
from jax.experimental import pallas as pl
from jax.experimental.pallas import tpu as pltpu
from jax.experimental.pallas import tpu_sc as plsc

_NUM_EXPERTS = 8
_MOE_DIM = 768
_FF_DIM = 1024
_BEH_DIM = 64
_BEH_PAD = 128
_IN_PAD = 896
_N_TOK = 2048
_TILE = 512
_P = 2616

_NC, _NS = 2, 16
_NW = _NC * _NS
_CHUNK = _N_TOK // _NW


def _dispatch(hidden, beh_emb, bidx, dst):
    mesh = plsc.VectorSubcoreMesh(core_axis_name="c", subcore_axis_name="s")

    @functools.partial(
        pl.kernel,
        mesh=mesh,
        out_type=(
            jax.ShapeDtypeStruct((_P, _MOE_DIM), jnp.float32),
            jax.ShapeDtypeStruct((_P, _BEH_PAD), jnp.float32),
        ),
        scratch_types=[
            pltpu.VMEM((_CHUNK,), jnp.int32),
            pltpu.VMEM((_CHUNK,), jnp.int32),
            pltpu.VMEM((_CHUNK, _MOE_DIM), jnp.float32),
            pltpu.VMEM((_CHUNK, _BEH_PAD), jnp.float32),
            pltpu.SemaphoreType.DMA,
            pltpu.SemaphoreType.DMA,
            pltpu.SemaphoreType.DMA,
        ],
    )
    def k(hidden_hbm, emb_hbm, bidx_hbm, dst_hbm, xh_hbm, xb_hbm,
          idx_v, bidx_v, hid_v, beh_v, sem_a, sem_b, sem_c):
        wid = lax.axis_index("s") * _NC + lax.axis_index("c")
        base = wid * _CHUNK
        cp_dst = pltpu.async_copy(dst_hbm.at[pl.ds(base, _CHUNK)], idx_v, sem_a)
        cp_bi = pltpu.async_copy(bidx_hbm.at[pl.ds(base, _CHUNK)], bidx_v,
                                 sem_b)
        cp_hid = pltpu.async_copy(hidden_hbm.at[pl.ds(base, _CHUNK)], hid_v,
                                  sem_c)
        cp_bi.wait()
        cp_emb = pltpu.async_copy(emb_hbm.at[bidx_v], beh_v, sem_b)
        cp_dst.wait()
        cp_hid.wait()
        cp_xh = pltpu.async_copy(hid_v, xh_hbm.at[idx_v], sem_c)
        cp_emb.wait()
        cp_xb = pltpu.async_copy(beh_v, xb_hbm.at[idx_v], sem_b)
        cp_xh.wait()
        cp_xb.wait()

    return k(hidden, beh_emb, bidx, dst)


def _group_gemm(xh, xb, wi, wo, scalars):
    def body(sc_ref, xh_ref, xb_ref, wi_hbm, wo_hbm, y_ref,
             wi_buf, wo_buf, wi_sem, wo_sem):
        def start_w(e, slot):
            pltpu.make_async_copy(wi_hbm.at[e], wi_buf.at[slot],
                                  wi_sem.at[slot]).start()
            pltpu.make_async_copy(wo_hbm.at[e], wo_buf.at[slot],
                                  wo_sem.at[slot]).start()

        def wait_w(e, slot):
            pltpu.make_async_copy(wi_hbm.at[e], wi_buf.at[slot],
                                  wi_sem.at[slot]).wait()
            pltpu.make_async_copy(wo_hbm.at[e], wo_buf.at[slot],
                                  wo_sem.at[slot]).wait()

        start_w(0, 0)
        start_w(1, 1)
        for e in range(_NUM_EXPERTS):
            slot = e % 3
            if e + 2 < _NUM_EXPERTS:
                start_w(e + 2, (e + 2) % 3)
            wait_w(e, slot)
            off = sc_ref[e]
            ntiles = sc_ref[_NUM_EXPERTS + e]
            wih = wi_buf[slot, :, :_MOE_DIM].astype(jnp.bfloat16)
            wib = wi_buf[slot, :, _MOE_DIM:_MOE_DIM + _BEH_DIM].astype(
                jnp.bfloat16)
            wout = wo_buf[slot].astype(jnp.bfloat16)

            def tile(k, carry):
                s = pl.multiple_of(off + k * _TILE, 8)
                xh16 = xh_ref[pl.ds(s, _TILE), :].astype(jnp.bfloat16)
                xb16 = xb_ref[pl.ds(s, _TILE), :_BEH_DIM].astype(jnp.bfloat16)
                acc = lax.dot_general(xh16, wih,
                                      (((1,), (1,)), ((), ())),
                                      preferred_element_type=jnp.float32)
                acc += lax.dot_general(xb16, wib,
                                       (((1,), (1,)), ((), ())),
                                       preferred_element_type=jnp.float32)
                inter = jnp.maximum(acc, 0.0).astype(jnp.bfloat16)
                y_ref[pl.ds(s, _TILE), :] = lax.dot_general(
                    inter, wout, (((1,), (1,)), ((), ())),
                    preferred_element_type=jnp.float32)
                return carry

            lax.fori_loop(0, ntiles, tile, 0)

    grid_spec = pltpu.PrefetchScalarGridSpec(
        num_scalar_prefetch=1,
        grid=(1,),
        in_specs=[
            pl.BlockSpec((_P, _MOE_DIM), lambda i, sc: (0, 0)),
            pl.BlockSpec((_P, _BEH_PAD), lambda i, sc: (0, 0)),
            pl.BlockSpec(memory_space=pl.ANY),
            pl.BlockSpec(memory_space=pl.ANY),
        ],
        out_specs=pl.BlockSpec((_P, _MOE_DIM), lambda i, sc: (0, 0)),
        scratch_shapes=[
            pltpu.VMEM((3, _FF_DIM, _MOE_DIM + _BEH_DIM), jnp.float32),
            pltpu.VMEM((3, _MOE_DIM, _FF_DIM), jnp.float32),
            pltpu.SemaphoreType.DMA((3,)),
            pltpu.SemaphoreType.DMA((3,)),
        ],
    )
    return pl.pallas_call(
        body,
        grid_spec=grid_spec,
        out_shape=jax.ShapeDtypeStruct((_P, _MOE_DIM), jnp.float32),
    )(scalars, xh, xb, wi, wo)


def _unsort(y_pad, dst):
    mesh = plsc.VectorSubcoreMesh(core_axis_name="c", subcore_axis_name="s")

    @functools.partial(
        pl.kernel,
        mesh=mesh,
        out_type=jax.ShapeDtypeStruct((_N_TOK, _MOE_DIM), jnp.float32),
        scratch_types=[
            pltpu.VMEM((_CHUNK,), jnp.int32),
            pltpu.VMEM((_CHUNK, _MOE_DIM), jnp.float32),
            pltpu.SemaphoreType.DMA,
        ],
    )
    def k(y_hbm, dst_hbm, o_hbm, idx_v, rows_v, sem):
        wid = lax.axis_index("s") * _NC + lax.axis_index("c")
        base = wid * _CHUNK
        pltpu.sync_copy(dst_hbm.at[pl.ds(base, _CHUNK)], idx_v)
        pltpu.async_copy(y_hbm.at[idx_v], rows_v, sem).wait()
        pltpu.sync_copy(rows_v, o_hbm.at[pl.ds(base, _CHUNK)])

    return k(y_pad, dst)


def _routing(pos):
    onehot = (pos[:, None] == jnp.arange(_NUM_EXPERTS, dtype=jnp.int32)[None, :]
              ).astype(jnp.int32)
    csum = jnp.cumsum(onehot, axis=0)
    counts = csum[-1]
    rank = jnp.sum(csum * onehot, axis=1) - 1
    pad8 = ((counts + 7) // 8) * 8
    off8 = jnp.cumsum(pad8) - pad8
    dst = (jnp.sum(off8[None, :] * onehot, axis=1) + rank).astype(jnp.int32)
    ntiles = (counts + _TILE - 1) // _TILE
    scalars = jnp.concatenate([off8, ntiles]).astype(jnp.int32)
    return dst, scalars


def kernel(hidden_states, position_index, behavior_index, Wi, Wo,
           behavior_embedding):
    pos = position_index.astype(jnp.int32)
    bidx = behavior_index.astype(jnp.int32)
    dst, scalars = _routing(pos)
    emb_pad = jnp.pad(behavior_embedding, ((0, 0), (0, _BEH_PAD - _BEH_DIM)))
    xh, xb = _dispatch(hidden_states, emb_pad, bidx, dst)
    y_pad = _group_gemm(xh, xb, Wi, Wo, scalars)
    return _unsort(y_pad, dst)


# speedup vs baseline: 1.0076x; 1.0076x over previous
import functools

import jax
import jax.numpy as jnp
from jax import lax
from jax.experimental import pallas as pl
from jax.experimental.pallas import tpu as pltpu
from jax.experimental.pallas import tpu_sc as plsc

_NUM_EXPERTS = 8
_MOE_DIM = 768
_FF_DIM = 1024
_BEH_DIM = 64
_BEH_PAD = 128
_IN_PAD = 896
_N_TOK = 2048
_TILE = 256
_P = 2360

_NC, _NS = 2, 16
_NW = _NC * _NS
_CHUNK = _N_TOK // _NW


def _dispatch(hidden, beh_emb, bidx, dst):
    mesh = plsc.VectorSubcoreMesh(core_axis_name="c", subcore_axis_name="s")

    @functools.partial(
        pl.kernel,
        mesh=mesh,
        out_type=(
            jax.ShapeDtypeStruct((_P, _MOE_DIM), jnp.float32),
            jax.ShapeDtypeStruct((_P, _BEH_PAD), jnp.float32),
        ),
        scratch_types=[
            pltpu.VMEM((_CHUNK,), jnp.int32),
            pltpu.VMEM((_CHUNK,), jnp.int32),
            pltpu.VMEM((_CHUNK, _MOE_DIM), jnp.float32),
            pltpu.VMEM((_CHUNK, _BEH_PAD), jnp.float32),
            pltpu.SemaphoreType.DMA,
            pltpu.SemaphoreType.DMA,
            pltpu.SemaphoreType.DMA,
        ],
    )
    def k(hidden_hbm, emb_hbm, bidx_hbm, dst_hbm, xh_hbm, xb_hbm,
          idx_v, bidx_v, hid_v, beh_v, sem_a, sem_b, sem_c):
        wid = lax.axis_index("s") * _NC + lax.axis_index("c")
        base = wid * _CHUNK
        cp_dst = pltpu.async_copy(dst_hbm.at[pl.ds(base, _CHUNK)], idx_v, sem_a)
        cp_bi = pltpu.async_copy(bidx_hbm.at[pl.ds(base, _CHUNK)], bidx_v,
                                 sem_b)
        cp_hid = pltpu.async_copy(hidden_hbm.at[pl.ds(base, _CHUNK)], hid_v,
                                  sem_c)
        cp_bi.wait()
        cp_emb = pltpu.async_copy(emb_hbm.at[bidx_v], beh_v, sem_b)
        cp_dst.wait()
        cp_hid.wait()
        cp_xh = pltpu.async_copy(hid_v, xh_hbm.at[idx_v], sem_c)
        cp_emb.wait()
        cp_xb = pltpu.async_copy(beh_v, xb_hbm.at[idx_v], sem_b)
        cp_xh.wait()
        cp_xb.wait()

    return k(hidden, beh_emb, bidx, dst)


def _group_gemm(xh, xb, wi, wo, scalars):
    def body(sc_ref, xh_ref, xb_ref, wi_hbm, wo_hbm, y_ref,
             wi_buf, wo_buf, wi_sem, wo_sem):
        def start_w(e, slot):
            pltpu.make_async_copy(wi_hbm.at[e], wi_buf.at[slot],
                                  wi_sem.at[slot]).start()
            pltpu.make_async_copy(wo_hbm.at[e], wo_buf.at[slot],
                                  wo_sem.at[slot]).start()

        def wait_w(e, slot):
            pltpu.make_async_copy(wi_hbm.at[e], wi_buf.at[slot],
                                  wi_sem.at[slot]).wait()
            pltpu.make_async_copy(wo_hbm.at[e], wo_buf.at[slot],
                                  wo_sem.at[slot]).wait()

        start_w(0, 0)
        start_w(1, 1)
        for e in range(_NUM_EXPERTS):
            slot = e % 3
            if e + 2 < _NUM_EXPERTS:
                start_w(e + 2, (e + 2) % 3)
            wait_w(e, slot)
            off = sc_ref[e]
            ntiles = sc_ref[_NUM_EXPERTS + e]
            wih = wi_buf[slot, :, :_MOE_DIM].astype(jnp.bfloat16)
            wib = wi_buf[slot, :, _MOE_DIM:_MOE_DIM + _BEH_DIM].astype(
                jnp.bfloat16)
            wout = wo_buf[slot].astype(jnp.bfloat16)

            def tile(k, carry):
                s = pl.multiple_of(off + k * _TILE, 8)
                xh16 = xh_ref[pl.ds(s, _TILE), :].astype(jnp.bfloat16)
                xb16 = xb_ref[pl.ds(s, _TILE), :_BEH_DIM].astype(jnp.bfloat16)
                acc = lax.dot_general(xh16, wih,
                                      (((1,), (1,)), ((), ())),
                                      preferred_element_type=jnp.float32)
                acc += lax.dot_general(xb16, wib,
                                       (((1,), (1,)), ((), ())),
                                       preferred_element_type=jnp.float32)
                inter = jnp.maximum(acc, 0.0).astype(jnp.bfloat16)
                y_ref[pl.ds(s, _TILE), :] = lax.dot_general(
                    inter, wout, (((1,), (1,)), ((), ())),
                    preferred_element_type=jnp.float32)
                return carry

            lax.fori_loop(0, ntiles, tile, 0)

    grid_spec = pltpu.PrefetchScalarGridSpec(
        num_scalar_prefetch=1,
        grid=(1,),
        in_specs=[
            pl.BlockSpec((_P, _MOE_DIM), lambda i, sc: (0, 0)),
            pl.BlockSpec((_P, _BEH_PAD), lambda i, sc: (0, 0)),
            pl.BlockSpec(memory_space=pl.ANY),
            pl.BlockSpec(memory_space=pl.ANY),
        ],
        out_specs=pl.BlockSpec((_P, _MOE_DIM), lambda i, sc: (0, 0)),
        scratch_shapes=[
            pltpu.VMEM((3, _FF_DIM, _MOE_DIM + _BEH_DIM), jnp.float32),
            pltpu.VMEM((3, _MOE_DIM, _FF_DIM), jnp.float32),
            pltpu.SemaphoreType.DMA((3,)),
            pltpu.SemaphoreType.DMA((3,)),
        ],
    )
    return pl.pallas_call(
        body,
        grid_spec=grid_spec,
        out_shape=jax.ShapeDtypeStruct((_P, _MOE_DIM), jnp.float32),
    )(scalars, xh, xb, wi, wo)


def _unsort(y_pad, dst):
    mesh = plsc.VectorSubcoreMesh(core_axis_name="c", subcore_axis_name="s")

    @functools.partial(
        pl.kernel,
        mesh=mesh,
        out_type=jax.ShapeDtypeStruct((_N_TOK, _MOE_DIM), jnp.float32),
        scratch_types=[
            pltpu.VMEM((_CHUNK,), jnp.int32),
            pltpu.VMEM((_CHUNK, _MOE_DIM), jnp.float32),
            pltpu.SemaphoreType.DMA,
        ],
    )
    def k(y_hbm, dst_hbm, o_hbm, idx_v, rows_v, sem):
        wid = lax.axis_index("s") * _NC + lax.axis_index("c")
        base = wid * _CHUNK
        pltpu.sync_copy(dst_hbm.at[pl.ds(base, _CHUNK)], idx_v)
        pltpu.async_copy(y_hbm.at[idx_v], rows_v, sem).wait()
        pltpu.sync_copy(rows_v, o_hbm.at[pl.ds(base, _CHUNK)])

    return k(y_pad, dst)


def _routing(pos):
    onehot = (pos[:, None] == jnp.arange(_NUM_EXPERTS, dtype=jnp.int32)[None, :]
              ).astype(jnp.int32)
    csum = jnp.cumsum(onehot, axis=0)
    counts = csum[-1]
    rank = jnp.sum(csum * onehot, axis=1) - 1
    pad8 = ((counts + 7) // 8) * 8
    off8 = jnp.cumsum(pad8) - pad8
    dst = (jnp.sum(off8[None, :] * onehot, axis=1) + rank).astype(jnp.int32)
    ntiles = (counts + _TILE - 1) // _TILE
    scalars = jnp.concatenate([off8, ntiles]).astype(jnp.int32)
    return dst, scalars


def kernel(hidden_states, position_index, behavior_index, Wi, Wo,
           behavior_embedding):
    pos = position_index.astype(jnp.int32)
    bidx = behavior_index.astype(jnp.int32)
    dst, scalars = _routing(pos)
    emb_pad = jnp.pad(behavior_embedding, ((0, 0), (0, _BEH_PAD - _BEH_DIM)))
    xh, xb = _dispatch(hidden_states, emb_pad, bidx, dst)
    y_pad = _group_gemm(xh, xb, Wi, Wo, scalars)
    return _unsort(y_pad, dst)


# speedup vs baseline: 1.0076x; 1.0000x over previous
import functools

import jax
import jax.numpy as jnp
from jax import lax
from jax.experimental import pallas as pl
from jax.experimental.pallas import tpu as pltpu
from jax.experimental.pallas import tpu_sc as plsc

_NUM_EXPERTS = 8
_MOE_DIM = 768
_FF_DIM = 1024
_BEH_DIM = 64
_BEH_PAD = 128
_N_TOK = 2048
_TILE = 256
_P = 2360

_NC, _NS = 2, 16
_NW = _NC * _NS
_CHUNK = _N_TOK // _NW


def _dispatch(hidden, beh_emb, bidx, dst):
    mesh = plsc.VectorSubcoreMesh(core_axis_name="c", subcore_axis_name="s")

    @functools.partial(
        pl.kernel,
        mesh=mesh,
        out_type=(
            jax.ShapeDtypeStruct((_P, _MOE_DIM), jnp.float32),
            jax.ShapeDtypeStruct((_P, _BEH_PAD), jnp.float32),
        ),
        scratch_types=[
            pltpu.VMEM((_CHUNK,), jnp.int32),
            pltpu.VMEM((_CHUNK,), jnp.int32),
            pltpu.VMEM((_CHUNK, _MOE_DIM), jnp.float32),
            pltpu.VMEM((_CHUNK, _BEH_PAD), jnp.float32),
            pltpu.SemaphoreType.DMA,
            pltpu.SemaphoreType.DMA,
            pltpu.SemaphoreType.DMA,
        ],
    )
    def k(hidden_hbm, emb_hbm, bidx_hbm, dst_hbm, xh_hbm, xb_hbm,
          idx_v, bidx_v, hid_v, beh_v, sem_a, sem_b, sem_c):
        wid = lax.axis_index("s") * _NC + lax.axis_index("c")
        base = wid * _CHUNK
        cp_dst = pltpu.async_copy(dst_hbm.at[pl.ds(base, _CHUNK)], idx_v, sem_a)
        cp_bi = pltpu.async_copy(bidx_hbm.at[pl.ds(base, _CHUNK)], bidx_v,
                                 sem_b)
        cp_hid = pltpu.async_copy(hidden_hbm.at[pl.ds(base, _CHUNK)], hid_v,
                                  sem_c)
        cp_bi.wait()
        cp_emb = pltpu.async_copy(emb_hbm.at[bidx_v], beh_v, sem_b)
        cp_dst.wait()
        cp_hid.wait()
        cp_xh = pltpu.async_copy(hid_v, xh_hbm.at[idx_v], sem_c)
        cp_emb.wait()
        cp_xb = pltpu.async_copy(beh_v, xb_hbm.at[idx_v], sem_b)
        cp_xh.wait()
        cp_xb.wait()

    return k(hidden, beh_emb, bidx, dst)


def _group_gemm(xh, xb, wi, wo, scalars):
    def body(sc_ref, xh_ref, xb_ref, wi_hbm, wo_hbm, y_ref,
             wi_buf, wo_buf, wi_sem, wo_sem):
        def start_w(e, slot):
            pltpu.make_async_copy(wi_hbm.at[e], wi_buf.at[slot],
                                  wi_sem.at[slot]).start()
            pltpu.make_async_copy(wo_hbm.at[e], wo_buf.at[slot],
                                  wo_sem.at[slot]).start()

        def wait_w(e, slot):
            pltpu.make_async_copy(wi_hbm.at[e], wi_buf.at[slot],
                                  wi_sem.at[slot]).wait()
            pltpu.make_async_copy(wo_hbm.at[e], wo_buf.at[slot],
                                  wo_sem.at[slot]).wait()

        start_w(0, 0)
        start_w(1, 1)
        for e in range(_NUM_EXPERTS):
            slot = e % 3
            if e + 2 < _NUM_EXPERTS:
                start_w(e + 2, (e + 2) % 3)
            wait_w(e, slot)
            off = sc_ref[e]
            ntiles = sc_ref[_NUM_EXPERTS + e]
            wih = wi_buf[slot, :, :_MOE_DIM].astype(jnp.bfloat16)
            wib = wi_buf[slot, :, _MOE_DIM:_MOE_DIM + _BEH_DIM].astype(
                jnp.bfloat16)
            wout = wo_buf[slot].astype(jnp.bfloat16)

            def tile(k, carry):
                s = pl.multiple_of(off + k * _TILE, 8)
                xh16 = xh_ref[pl.ds(s, _TILE), :].astype(jnp.bfloat16)
                xb16 = xb_ref[pl.ds(s, _TILE), :_BEH_DIM].astype(jnp.bfloat16)
                acc = lax.dot_general(xh16, wih,
                                      (((1,), (1,)), ((), ())),
                                      preferred_element_type=jnp.float32)
                acc += lax.dot_general(xb16, wib,
                                       (((1,), (1,)), ((), ())),
                                       preferred_element_type=jnp.float32)
                inter = jnp.maximum(acc, 0.0).astype(jnp.bfloat16)
                y_ref[pl.ds(s, _TILE), :] = lax.dot_general(
                    inter, wout, (((1,), (1,)), ((), ())),
                    preferred_element_type=jnp.float32)
                return carry

            lax.fori_loop(0, ntiles, tile, 0)

    grid_spec = pltpu.PrefetchScalarGridSpec(
        num_scalar_prefetch=1,
        grid=(1,),
        in_specs=[
            pl.BlockSpec((_P, _MOE_DIM), lambda i, sc: (0, 0)),
            pl.BlockSpec((_P, _BEH_PAD), lambda i, sc: (0, 0)),
            pl.BlockSpec(memory_space=pl.ANY),
            pl.BlockSpec(memory_space=pl.ANY),
        ],
        out_specs=pl.BlockSpec((_P, _MOE_DIM), lambda i, sc: (0, 0)),
        scratch_shapes=[
            pltpu.VMEM((3, _FF_DIM, _MOE_DIM + _BEH_DIM), jnp.float32),
            pltpu.VMEM((3, _MOE_DIM, _FF_DIM), jnp.float32),
            pltpu.SemaphoreType.DMA((3,)),
            pltpu.SemaphoreType.DMA((3,)),
        ],
    )
    return pl.pallas_call(
        body,
        grid_spec=grid_spec,
        out_shape=jax.ShapeDtypeStruct((_P, _MOE_DIM), jnp.float32),
    )(scalars, xh, xb, wi, wo)


def _unsort(y_pad, dst):
    mesh = plsc.VectorSubcoreMesh(core_axis_name="c", subcore_axis_name="s")

    @functools.partial(
        pl.kernel,
        mesh=mesh,
        out_type=jax.ShapeDtypeStruct((_N_TOK, _MOE_DIM), jnp.float32),
        scratch_types=[
            pltpu.VMEM((_CHUNK,), jnp.int32),
            pltpu.VMEM((_CHUNK, _MOE_DIM), jnp.float32),
            pltpu.SemaphoreType.DMA,
        ],
    )
    def k(y_hbm, dst_hbm, o_hbm, idx_v, rows_v, sem):
        wid = lax.axis_index("s") * _NC + lax.axis_index("c")
        base = wid * _CHUNK
        pltpu.sync_copy(dst_hbm.at[pl.ds(base, _CHUNK)], idx_v)
        pltpu.async_copy(y_hbm.at[idx_v], rows_v, sem).wait()
        pltpu.sync_copy(rows_v, o_hbm.at[pl.ds(base, _CHUNK)])

    return k(y_pad, dst)


def _routing(pos):
    onehot = (pos[:, None] == jnp.arange(_NUM_EXPERTS, dtype=jnp.int32)[None, :]
              ).astype(jnp.int32)
    csum = jnp.cumsum(onehot, axis=0)
    counts = csum[-1]
    rank = jnp.sum(csum * onehot, axis=1) - 1
    pad8 = ((counts + 7) // 8) * 8
    off8 = jnp.cumsum(pad8) - pad8
    dst = (jnp.sum(off8[None, :] * onehot, axis=1) + rank).astype(jnp.int32)
    ntiles = (counts + _TILE - 1) // _TILE
    scalars = jnp.concatenate([off8, ntiles]).astype(jnp.int32)
    return dst, scalars


def kernel(hidden_states, position_index, behavior_index, Wi, Wo,
           behavior_embedding):
    pos = position_index.astype(jnp.int32)
    bidx = behavior_index.astype(jnp.int32)
    dst, scalars = _routing(pos)
    emb_pad = jnp.pad(behavior_embedding, ((0, 0), (0, _BEH_PAD - _BEH_DIM)))
    xh, xb = _dispatch(hidden_states, emb_pad, bidx, dst)
    y_pad = _group_gemm(xh, xb, Wi, Wo, scalars)
    return _unsort(y_pad, dst)
